# baseline (speedup 1.0000x reference)
import functools

import numpy as np

import jax
import jax.numpy as jnp
from jax import lax
from jax.experimental import pallas as pl
from jax.experimental.pallas import tpu as pltpu
from jax.experimental.pallas import tpu_sc as plsc

MUL0 = 16
MUL1 = 8
N_NODES = 10000
N_EDGES = 160000
EMB_IN = 96
W_NUMEL = 768

EB = 3200
MSG_D = 64
CHUNK = 128
ROWS_PER_W = 40
GRP = 4
EPAD = 32 * ROWS_PER_W * CHUNK
IDX_ROWS = EPAD // CHUNK
N_PAD = 10016
NZ = N_PAD // 16

_f32 = jnp.float32
_I0 = np.int32(0)


def _im_row(i):
    return (i, _I0)


def _im_zero2(i):
    return (_I0, _I0)


def _im_zero3(i):
    return (_I0, _I0, _I0)


def _dot(a, b):
    return jnp.dot(a.astype(jnp.bfloat16), b.astype(jnp.bfloat16),
                   preferred_element_type=jnp.float32)


def _iota2(shape):
    r = lax.broadcasted_iota(jnp.int32, shape, 0)
    c = lax.broadcasted_iota(jnp.int32, shape, 1)
    return r, c


def _repmat(n_in, rep):
    r, c = _iota2((n_in, n_in * rep))
    return (c // rep == r).astype(_f32)


def _summat(n_in, rep):
    r, c = _iota2((n_in * rep, rep))
    return (r % rep == c).astype(_f32)


def _sc_gather(t16, t48, dst_rows, src_rows):
    mesh = plsc.VectorSubcoreMesh(core_axis_name="c", subcore_axis_name="s")

    @functools.partial(
        pl.kernel,
        out_type=[jax.ShapeDtypeStruct((EPAD, 16), _f32),
                  jax.ShapeDtypeStruct((EPAD, 48), _f32)],
        mesh=mesh,
        scratch_types=[pltpu.VMEM((ROWS_PER_W, CHUNK), jnp.int32),
                       pltpu.VMEM((ROWS_PER_W, CHUNK), jnp.int32),
                       pltpu.VMEM((2, GRP, CHUNK, 16), _f32),
                       pltpu.VMEM((2, GRP, CHUNK, 48), _f32),
                       pltpu.SemaphoreType.DMA,
                       pltpu.SemaphoreType.DMA],
        compiler_params=pltpu.CompilerParams(use_tc_tiling_on_sc=False),
    )
    def gk(t16_hbm, t48_hbm, dst_hbm, src_hbm, xdg_hbm, xsg_hbm,
           idxd_v, idxs_v, r16_v, r48_v, sem1, sem2):
        wid = lax.axis_index("s") * 2 + lax.axis_index("c")
        rbase = wid * ROWS_PER_W
        pltpu.sync_copy(dst_hbm.at[pl.ds(rbase, ROWS_PER_W)], idxd_v)
        pltpu.sync_copy(src_hbm.at[pl.ds(rbase, ROWS_PER_W)], idxs_v)

        ngrp = ROWS_PER_W // GRP

        def fire(k, half):
            hs = []
            h32 = np.int32(half)
            for b in range(GRP):
                j = np.int32(k * GRP + b)
                b32 = np.int32(b)
                hs.append(pltpu.async_copy(
                    t16_hbm.at[idxd_v.at[j]], r16_v.at[h32, b32], sem1))
                hs.append(pltpu.async_copy(
                    t48_hbm.at[idxs_v.at[j]], r48_v.at[h32, b32], sem2))
            return hs

        hs = fire(0, 0)
        for k in range(ngrp):
            nxt = None
            if k + 1 < ngrp:
                nxt = fire(k + 1, (k + 1) % 2)
            for h in hs:
                h.wait()
            for b in range(GRP):
                eb = (rbase + np.int32(k * GRP + b)) * np.int32(CHUNK)
                pltpu.sync_copy(r16_v.at[np.int32(k % 2), np.int32(b)],
                                xdg_hbm.at[pl.ds(eb, CHUNK)])
                pltpu.sync_copy(r48_v.at[np.int32(k % 2), np.int32(b)],
                                xsg_hbm.at[pl.ds(eb, CHUNK)])
            hs = nxt

    return gk(t16, t48, dst_rows, src_rows)


def _edge_body(xd_ref, xs_ref, ea_ref, y_ref, W_ref, out_ref):
    gar, gac = _iota2((24, 24))
    gall = ((gar % 3) * 8 + gar // 3 == gac).astype(_f32)

    segs = ((0, 0, 16, 16, 0), (16, 256, 8, 16, 16), (24, 384, 16, 8, 32),
            (40, 512, 8, 8, 40), (48, 576, 16, 8, 48), (64, 704, 8, 8, 56),
            (72, 768, 8, 8, 64), (80, 832, 8, 8, 72))
    rr, rc = _iota2((88, 896))
    rbig = jnp.zeros((88, 896), _f32)
    sr, sc = _iota2((896, 80))
    sbig = jnp.zeros((896, 80), _f32)
    for ro, co, n, rep, oo in segs:
        span = n * rep
        rbig = rbig + (((rc >= co) & (rc < co + span)
                        & (rr == ro + (rc - co) // rep)).astype(_f32))
        sbig = sbig + (((sr >= co) & (sr < co + span)
                        & (sc >= oo) & (sc < oo + rep)
                        & ((sr - co) % rep == sc - oo)).astype(_f32))

    xd16 = xd_ref[...]
    xs48 = xs_ref[...]
    ea = ea_ref[...]
    y = y_ref[...]
    xs = xs48[:, :16]
    xv = xs48[:, 16:40]
    y0 = y[:, 0:1]
    y1 = y[:, 1:4]

    emb = jnp.concatenate([xd16, xs, ea], axis=1)
    w = _dot(emb, W_ref[...])
    col = lax.broadcasted_iota(jnp.int32, (1, W_NUMEL), 1)
    in_dot = ((col >= 256) & (col < 384)) | ((col >= 512) & (col < 576))
    base = np.float32(1.0 / (np.sqrt(24.0) * np.sqrt(96.0)))
    w = w * jnp.where(in_dot, np.float32(base / np.sqrt(3.0)), base)

    xvall = _dot(xv, gall)
    dotr = (xvall[:, 0:8] * y1[:, 0:1] + xvall[:, 8:16] * y1[:, 1:2]
            + xvall[:, 16:24] * y1[:, 2:3])
    xsy0 = xs * y0

    a = jnp.concatenate([xsy0, dotr, xsy0, dotr, xs, xvall], axis=1)
    arep = _dot(a, rbig)
    w6 = w[:, 704:768]
    wbig = jnp.concatenate([w, w6, w6], axis=1)
    m_all = _dot(arep * wbig, sbig)

    out_s = m_all[:, 0:16] + m_all[:, 16:32]
    out_g = m_all[:, 32:40] + m_all[:, 40:48]
    t5 = m_all[:, 48:56]
    vs = [y1[:, m:m + 1] * t5 + y0 * m_all[:, 56 + 8 * m:64 + 8 * m]
          for m in range(3)]

    B = xd16.shape[0]
    ones = jnp.ones((B, 1), _f32)
    zeros = jnp.zeros((B, MSG_D - 49), _f32)
    out_ref[...] = jnp.concatenate(
        [out_s, out_g, vs[0], vs[1], vs[2], ones, zeros], axis=1)


def _edge_messages(xdg, xsg, edge_attr, Yij, W_emb):
    n_blocks = N_EDGES // EB
    return pl.pallas_call(
        _edge_body,
        grid=(n_blocks,),
        in_specs=[
            pl.BlockSpec((EB, 16), _im_row),
            pl.BlockSpec((EB, 48), _im_row),
            pl.BlockSpec((EB, 64), _im_row),
            pl.BlockSpec((EB, 4), _im_row),
            pl.BlockSpec((EMB_IN, W_NUMEL), _im_zero2),
        ],
        out_specs=pl.BlockSpec((EB, MSG_D), _im_row),
        out_shape=jax.ShapeDtypeStruct((EPAD, MSG_D), _f32),
    )(xdg, xsg, edge_attr, Yij, W_emb)


def _sc_scatter(msg, dst_rows, zero64):
    mesh = plsc.VectorSubcoreMesh(core_axis_name="c", subcore_axis_name="s")
    nz = NZ

    @functools.partial(
        pl.kernel,
        out_type=jax.ShapeDtypeStruct((2, N_PAD, MSG_D), _f32),
        mesh=mesh,
        scratch_types=[pltpu.VMEM((ROWS_PER_W, CHUNK), jnp.int32),
                       pltpu.VMEM((2, GRP, CHUNK, MSG_D), _f32),
                       pltpu.VMEM_SHARED((N_PAD, MSG_D), _f32),
                       pltpu.SemaphoreType.DMA],
        compiler_params=pltpu.CompilerParams(use_tc_tiling_on_sc=False),
    )
    def sk(msg_hbm, dst_hbm, zero_hbm, out_hbm, idx_v, buf_v, acc_sh, sem):
        c = lax.axis_index("c")
        s = lax.axis_index("s")
        pltpu.sync_copy(zero_hbm.at[pl.ds(s * nz, nz)],
                        acc_sh.at[pl.ds(s * nz, nz)])
        plsc.subcore_barrier()
        rbase = c * (IDX_ROWS // 2) + s * ROWS_PER_W
        pltpu.sync_copy(dst_hbm.at[pl.ds(rbase, ROWS_PER_W)], idx_v)

        ngrp = ROWS_PER_W // GRP

        def fire(k, half):
            hs = []
            h32 = np.int32(half)
            for b in range(GRP):
                j = np.int32(k * GRP + b)
                hs.append(pltpu.async_copy(
                    msg_hbm.at[pl.ds((rbase + j) * np.int32(CHUNK), CHUNK)],
                    buf_v.at[h32, np.int32(b)], sem))
            return hs

        hs = fire(0, 0)
        for k in range(ngrp):
            nxt = fire(k + 1, (k + 1) % 2) if k + 1 < ngrp else None
            for h in hs:
                h.wait()
            for b in range(GRP):
                j = np.int32(k * GRP + b)
                pltpu.sync_copy(buf_v.at[np.int32(k % 2), np.int32(b)],
                                acc_sh.at[idx_v.at[j]], add=True)
            hs = nxt
        plsc.subcore_barrier()
        pltpu.sync_copy(acc_sh.at[pl.ds(s * nz, nz)],
                        out_hbm.at[c, pl.ds(s * nz, nz)])

    return sk(msg, dst_rows, zero64)


def _node_body(p_ref, x_ref, out_ref):
    s64 = p_ref[0] + p_ref[1]
    cnt = s64[:, 48:49]
    mean = s64[:, :48] / jnp.maximum(cnt, jnp.float32(1.0))
    s = jnp.maximum(mean[:, :16], 0.0)
    g = jnp.maximum(mean[:, 16:24], 0.0)
    pr, pc = _iota2((8, 24))
    acc = None
    for m in range(3):
        pm = ((pc // 3 == pr) & (pc % 3 == m)).astype(_f32)
        t = _dot(mean[:, 24 + 8 * m:32 + 8 * m] * g, pm)
        acc = t if acc is None else acc + t
    out_ref[...] = x_ref[...] + jnp.concatenate([s, acc], axis=1)


def _node_epilogue(partials, x):
    return pl.pallas_call(
        _node_body,
        grid=(1,),
        in_specs=[
            pl.BlockSpec((2, N_NODES, MSG_D), _im_zero3),
            pl.BlockSpec((N_NODES, 40), _im_zero2),
        ],
        out_specs=pl.BlockSpec((N_NODES, 40), _im_zero2),
        out_shape=jax.ShapeDtypeStruct((N_NODES, 40), _f32),
    )(partials, x)


def kernel(x, edge_attr, Yij, W_emb, edge_index):
    x = x.astype(_f32)
    edge_attr = edge_attr.astype(_f32)
    Yij = Yij.astype(_f32)
    W_emb = W_emb.astype(_f32)

    dst = edge_index[0].astype(jnp.int32)
    src = edge_index[1].astype(jnp.int32)
    pad_d = jnp.full((EPAD - N_EDGES,), N_NODES, jnp.int32)
    pad_s = jnp.zeros((EPAD - N_EDGES,), jnp.int32)
    dst_rows = jnp.concatenate([dst, pad_d]).reshape(IDX_ROWS, CHUNK)
    src_rows = jnp.concatenate([src, pad_s]).reshape(IDX_ROWS, CHUNK)

    t16 = jnp.pad(x[:, :16], ((0, N_PAD - N_NODES), (0, 0)))
    t48 = jnp.pad(x, ((0, N_PAD - N_NODES), (0, 8)))

    xdg, xsg = _sc_gather(t16, t48, dst_rows, src_rows)

    msg = _edge_messages(xdg, xsg, edge_attr, Yij, W_emb)

    zero64 = jnp.zeros((N_PAD, MSG_D), _f32)
    partials = _sc_scatter(msg, dst_rows, zero64)

    return _node_epilogue(partials, x)


# speedup vs baseline: 1.3024x; 1.3024x over previous
import functools

import numpy as np

import jax
import jax.numpy as jnp
from jax import lax
from jax.experimental import pallas as pl
from jax.experimental.pallas import tpu as pltpu
from jax.experimental.pallas import tpu_sc as plsc

MUL0 = 16
MUL1 = 8
N_NODES = 10000
N_EDGES = 160000
EMB_IN = 96
W_NUMEL = 768

EB = 4000
MSG_D = 64
CHUNK = 128
ROWS_PER_W = 40
GRP = 4
EPAD = 32 * ROWS_PER_W * CHUNK
IDX_ROWS = EPAD // CHUNK
N_PAD = 10016
NZ = N_PAD // 16

_f32 = jnp.float32
_I0 = np.int32(0)


def _im_row(i):
    return (i, _I0)


def _im_zero2(i):
    return (_I0, _I0)


def _im_zero3(i):
    return (_I0, _I0, _I0)


def _dot(a, b):
    return jnp.dot(a.astype(jnp.bfloat16), b.astype(jnp.bfloat16),
                   preferred_element_type=jnp.float32)


def _iota2(shape):
    r = lax.broadcasted_iota(jnp.int32, shape, 0)
    c = lax.broadcasted_iota(jnp.int32, shape, 1)
    return r, c


def _repmat(n_in, rep):
    r, c = _iota2((n_in, n_in * rep))
    return (c // rep == r).astype(_f32)


def _summat(n_in, rep):
    r, c = _iota2((n_in * rep, rep))
    return (r % rep == c).astype(_f32)


def _sc_gather(t16, t48, dst_rows, src_rows):
    mesh = plsc.VectorSubcoreMesh(core_axis_name="c", subcore_axis_name="s")

    @functools.partial(
        pl.kernel,
        out_type=[jax.ShapeDtypeStruct((EPAD, 16), _f32),
                  jax.ShapeDtypeStruct((EPAD, 48), _f32)],
        mesh=mesh,
        scratch_types=[pltpu.VMEM((ROWS_PER_W, CHUNK), jnp.int32),
                       pltpu.VMEM((ROWS_PER_W, CHUNK), jnp.int32),
                       pltpu.VMEM((2, GRP, CHUNK, 16), _f32),
                       pltpu.VMEM((2, GRP, CHUNK, 48), _f32),
                       pltpu.SemaphoreType.DMA,
                       pltpu.SemaphoreType.DMA],
        compiler_params=pltpu.CompilerParams(use_tc_tiling_on_sc=False),
    )
    def gk(t16_hbm, t48_hbm, dst_hbm, src_hbm, xdg_hbm, xsg_hbm,
           idxd_v, idxs_v, r16_v, r48_v, sem1, sem2):
        wid = lax.axis_index("s") * 2 + lax.axis_index("c")
        rbase = wid * ROWS_PER_W
        pltpu.sync_copy(dst_hbm.at[pl.ds(rbase, ROWS_PER_W)], idxd_v)
        pltpu.sync_copy(src_hbm.at[pl.ds(rbase, ROWS_PER_W)], idxs_v)

        ngrp = ROWS_PER_W // GRP

        def fire(k, half):
            hs = []
            h32 = np.int32(half)
            for b in range(GRP):
                j = np.int32(k * GRP + b)
                b32 = np.int32(b)
                hs.append(pltpu.async_copy(
                    t16_hbm.at[idxd_v.at[j]], r16_v.at[h32, b32], sem1))
                hs.append(pltpu.async_copy(
                    t48_hbm.at[idxs_v.at[j]], r48_v.at[h32, b32], sem2))
            return hs

        hs = fire(0, 0)
        for k in range(ngrp):
            nxt = None
            if k + 1 < ngrp:
                nxt = fire(k + 1, (k + 1) % 2)
            for h in hs:
                h.wait()
            for b in range(GRP):
                eb = (rbase + np.int32(k * GRP + b)) * np.int32(CHUNK)
                pltpu.sync_copy(r16_v.at[np.int32(k % 2), np.int32(b)],
                                xdg_hbm.at[pl.ds(eb, CHUNK)])
                pltpu.sync_copy(r48_v.at[np.int32(k % 2), np.int32(b)],
                                xsg_hbm.at[pl.ds(eb, CHUNK)])
            hs = nxt

    return gk(t16, t48, dst_rows, src_rows)


def _edge_body(xd_ref, xs_ref, ea_ref, y_ref, W_ref, out_ref):
    t3r, t3c = _iota2((3, 24))
    t3 = (t3c % 3 == t3r).astype(_f32)
    s24r, s24c = _iota2((24, 8))
    s24 = (s24r // 3 == s24c).astype(_f32)
    r1616 = _repmat(16, 16)
    s256 = _summat(16, 16)
    r816 = _repmat(8, 16)
    s12816 = _summat(8, 16)
    r168 = _repmat(16, 8)
    s1288 = _summat(16, 8)
    r88 = _repmat(8, 8)
    s648 = _summat(8, 8)

    xd16 = xd_ref[...]
    xs48 = xs_ref[...]
    ea = ea_ref[...]
    y = y_ref[...]
    xs = xs48[:, :16]
    xv = xs48[:, 16:40]
    y0 = y[:, 0:1]
    y1 = y[:, 1:4]

    emb = jnp.concatenate([xd16, xs, ea], axis=1)
    w = jnp.dot(emb.astype(jnp.bfloat16), W_ref[...].astype(jnp.bfloat16),
                preferred_element_type=jnp.float32)
    col = lax.broadcasted_iota(jnp.int32, (1, W_NUMEL), 1)
    in_dot = ((col >= 256) & (col < 384)) | ((col >= 512) & (col < 576))
    base = np.float32(1.0 / (np.sqrt(24.0) * np.sqrt(96.0)))
    w = w * jnp.where(in_dot, np.float32(base / np.sqrt(3.0)), base)

    y1rep = _dot(y1, t3)
    dotr = _dot(xv * y1rep, s24)
    xsy0 = xs * y0

    c1 = _dot(_dot(xsy0, r1616) * w[:, 0:256], s256)
    c2 = _dot(_dot(dotr, r816) * w[:, 256:384], s12816)
    out_s = c1 + c2

    c3 = _dot(_dot(xsy0, r168) * w[:, 384:512], s1288)
    c4 = _dot(_dot(dotr, r88) * w[:, 512:576], s648)
    out_g = c3 + c4

    t5 = _dot(_dot(xs, r168) * w[:, 576:704], s1288)
    w6 = w[:, 704:768]
    gr, gc = _iota2((24, 8))
    vs = []
    for m in range(3):
        gm = ((gr % 3 == m) & (gr // 3 == gc)).astype(_f32)
        xvm = _dot(xv, gm)
        t6 = _dot(_dot(xvm, r88) * w6, s648)
        vs.append(y1[:, m:m + 1] * t5 + y0 * t6)

    B = xd16.shape[0]
    ones = jnp.ones((B, 1), _f32)
    zeros = jnp.zeros((B, MSG_D - 49), _f32)
    out_ref[...] = jnp.concatenate(
        [out_s, out_g, vs[0], vs[1], vs[2], ones, zeros], axis=1)


def _edge_messages(xdg, xsg, edge_attr, Yij, W_emb):
    n_blocks = N_EDGES // EB
    return pl.pallas_call(
        _edge_body,
        grid=(n_blocks,),
        in_specs=[
            pl.BlockSpec((EB, 16), _im_row),
            pl.BlockSpec((EB, 48), _im_row),
            pl.BlockSpec((EB, 64), _im_row),
            pl.BlockSpec((EB, 4), _im_row),
            pl.BlockSpec((EMB_IN, W_NUMEL), _im_zero2),
        ],
        out_specs=pl.BlockSpec((EB, MSG_D), _im_row),
        out_shape=jax.ShapeDtypeStruct((EPAD, MSG_D), _f32),
    )(xdg, xsg, edge_attr, Yij, W_emb)


def _sc_scatter(msg, dst_rows, zero64):
    mesh = plsc.VectorSubcoreMesh(core_axis_name="c", subcore_axis_name="s")
    nz = NZ

    @functools.partial(
        pl.kernel,
        out_type=jax.ShapeDtypeStruct((2, N_PAD, MSG_D), _f32),
        mesh=mesh,
        scratch_types=[pltpu.VMEM((ROWS_PER_W, CHUNK), jnp.int32),
                       pltpu.VMEM((2, GRP, CHUNK, MSG_D), _f32),
                       pltpu.VMEM_SHARED((N_PAD, MSG_D), _f32),
                       pltpu.SemaphoreType.DMA],
        compiler_params=pltpu.CompilerParams(use_tc_tiling_on_sc=False),
    )
    def sk(msg_hbm, dst_hbm, zero_hbm, out_hbm, idx_v, buf_v, acc_sh, sem):
        c = lax.axis_index("c")
        s = lax.axis_index("s")
        pltpu.sync_copy(zero_hbm.at[pl.ds(s * nz, nz)],
                        acc_sh.at[pl.ds(s * nz, nz)])
        plsc.subcore_barrier()
        rbase = c * (IDX_ROWS // 2) + s * ROWS_PER_W
        pltpu.sync_copy(dst_hbm.at[pl.ds(rbase, ROWS_PER_W)], idx_v)

        ngrp = ROWS_PER_W // GRP

        def fire(k, half):
            hs = []
            h32 = np.int32(half)
            for b in range(GRP):
                j = np.int32(k * GRP + b)
                hs.append(pltpu.async_copy(
                    msg_hbm.at[pl.ds((rbase + j) * np.int32(CHUNK), CHUNK)],
                    buf_v.at[h32, np.int32(b)], sem))
            return hs

        hs = fire(0, 0)
        for k in range(ngrp):
            nxt = fire(k + 1, (k + 1) % 2) if k + 1 < ngrp else None
            for h in hs:
                h.wait()
            for b in range(GRP):
                j = np.int32(k * GRP + b)
                pltpu.sync_copy(buf_v.at[np.int32(k % 2), np.int32(b)],
                                acc_sh.at[idx_v.at[j]], add=True)
            hs = nxt
        plsc.subcore_barrier()
        pltpu.sync_copy(acc_sh.at[pl.ds(s * nz, nz)],
                        out_hbm.at[c, pl.ds(s * nz, nz)])

    return sk(msg, dst_rows, zero64)


def _node_body(p_ref, x_ref, out_ref):
    s64 = p_ref[0] + p_ref[1]
    cnt = s64[:, 48:49]
    mean = s64[:, :48] / jnp.maximum(cnt, jnp.float32(1.0))
    s = jnp.maximum(mean[:, :16], 0.0)
    g = jnp.maximum(mean[:, 16:24], 0.0)
    pr, pc = _iota2((8, 24))
    acc = None
    for m in range(3):
        pm = ((pc // 3 == pr) & (pc % 3 == m)).astype(_f32)
        t = _dot(mean[:, 24 + 8 * m:32 + 8 * m] * g, pm)
        acc = t if acc is None else acc + t
    out_ref[...] = x_ref[...] + jnp.concatenate([s, acc], axis=1)


def _node_epilogue(partials, x):
    return pl.pallas_call(
        _node_body,
        grid=(1,),
        in_specs=[
            pl.BlockSpec((2, N_NODES, MSG_D), _im_zero3),
            pl.BlockSpec((N_NODES, 40), _im_zero2),
        ],
        out_specs=pl.BlockSpec((N_NODES, 40), _im_zero2),
        out_shape=jax.ShapeDtypeStruct((N_NODES, 40), _f32),
    )(partials, x)


def kernel(x, edge_attr, Yij, W_emb, edge_index):
    x = x.astype(_f32)
    edge_attr = edge_attr.astype(_f32)
    Yij = Yij.astype(_f32)
    W_emb = W_emb.astype(_f32)

    dst = edge_index[0].astype(jnp.int32)
    src = edge_index[1].astype(jnp.int32)
    pad_d = jnp.full((EPAD - N_EDGES,), N_NODES, jnp.int32)
    pad_s = jnp.zeros((EPAD - N_EDGES,), jnp.int32)
    dst_rows = jnp.concatenate([dst, pad_d]).reshape(IDX_ROWS, CHUNK)
    src_rows = jnp.concatenate([src, pad_s]).reshape(IDX_ROWS, CHUNK)

    t16 = jnp.pad(x[:, :16], ((0, N_PAD - N_NODES), (0, 0)))
    t48 = jnp.pad(x, ((0, N_PAD - N_NODES), (0, 8)))

    xdg, xsg = _sc_gather(t16, t48, dst_rows, src_rows)

    msg = _edge_messages(xdg, xsg, edge_attr, Yij, W_emb)

    zero64 = jnp.zeros((N_PAD, MSG_D), _f32)
    partials = _sc_scatter(msg, dst_rows, zero64)

    return _node_epilogue(partials, x)
